# baseline (speedup 1.0000x reference)
import functools

import jax
import jax.numpy as jnp
from jax import lax
from jax.experimental import pallas as pl
from jax.experimental.pallas import tpu as pltpu
from jax.experimental.pallas import tpu_sc as plsc

_LANES = 16
_CHUNK = 128
_WIDE = 128


def _widen_rows(table):
    v, d = table.shape
    rows_per_blk = 2000
    n_blk = v // rows_per_blk

    def body(in_ref, out_ref):
        out_ref[:, 0:d] = in_ref[...]

    return pl.pallas_call(
        body,
        grid=(n_blk,),
        in_specs=[pl.BlockSpec((rows_per_blk, d), lambda i: (i, 0))],
        out_specs=pl.BlockSpec((rows_per_blk, _WIDE), lambda i: (i, 0)),
        out_shape=jax.ShapeDtypeStruct((v, _WIDE), jnp.float32),
    )(table)


@functools.partial(jax.jit, static_argnames=("num_cores", "num_subcores"))
def _w2v_scores(target2d, context2d, table, *, num_cores, num_subcores):
    n_chunks, chunk = target2d.shape
    batch = n_chunks * chunk
    _, wide = table.shape
    embed = 64
    num_workers = num_cores * num_subcores
    b_per_w = batch // num_workers
    chunks_per_w = b_per_w // chunk
    half = b_per_w // 2

    mesh = plsc.VectorSubcoreMesh(core_axis_name="c", subcore_axis_name="s")

    @functools.partial(
        pl.kernel,
        mesh=mesh,
        out_type=jax.ShapeDtypeStruct((batch,), jnp.float32),
        scratch_types=[
            pltpu.VMEM((chunks_per_w, chunk), jnp.int32),
            pltpu.VMEM((chunks_per_w, chunk), jnp.int32),
            pltpu.VMEM((half, wide), jnp.float32),
            pltpu.VMEM((half, wide), jnp.float32),
            pltpu.VMEM((b_per_w,), jnp.float32),
            pltpu.SemaphoreType.DMA,
        ],
        compiler_params=pltpu.CompilerParams(
            needs_layout_passes=False, use_tc_tiling_on_sc=True),
    )
    def k(tgt_hbm, ctx_hbm, table_hbm, out_hbm, tidx_v, cidx_v, trows_v,
          crows_v, out_v, sem):
        wid = lax.axis_index("s") * num_cores + lax.axis_index("c")
        base = wid * b_per_w
        cbase = wid * chunks_per_w

        pltpu.sync_copy(tgt_hbm.at[pl.ds(cbase, chunks_per_w)], tidx_v)
        pltpu.sync_copy(ctx_hbm.at[pl.ds(cbase, chunks_per_w)], cidx_v)

        lane_iota = lax.iota(jnp.int32, _LANES)
        n_sub = embed // _LANES
        chunks_per_half = half // chunk

        for h in range(2):
            copies = []
            for j in range(chunks_per_half):
                cj = h * chunks_per_half + j
                copies.append(pltpu.async_copy(
                    table_hbm.at[tidx_v.at[cj]],
                    trows_v.at[pl.ds(j * chunk, chunk)], sem))
                copies.append(pltpu.async_copy(
                    table_hbm.at[cidx_v.at[cj]],
                    crows_v.at[pl.ds(j * chunk, chunk)], sem))
            for c in copies:
                c.wait()

            def group_body(g, _):
                base_r = g * _LANES
                res = jnp.zeros((_LANES,), jnp.float32)
                for r in range(_LANES):
                    row = base_r + r
                    acc = (trows_v[row, pl.ds(0, _LANES)] *
                           crows_v[row, pl.ds(0, _LANES)])
                    for j in range(1, n_sub):
                        acc = acc + (trows_v[row, pl.ds(j * _LANES, _LANES)] *
                                     crows_v[row, pl.ds(j * _LANES, _LANES)])
                    res = jnp.where(lane_iota == r, jnp.sum(acc), res)
                out_v[pl.ds(h * half + base_r, _LANES)] = res
                return 0

            lax.fori_loop(0, half // _LANES, group_body, 0)

        pltpu.sync_copy(out_v, out_hbm.at[pl.ds(base, b_per_w)])

    return k(target2d, context2d, table)


def kernel(target, context, word_embeddings):
    info = plsc.get_sparse_core_info()
    batch = target.shape[0]
    t2 = target.reshape(batch // _CHUNK, _CHUNK)
    c2 = context.reshape(batch // _CHUNK, _CHUNK)
    table_wide = _widen_rows(word_embeddings)
    return _w2v_scores(t2, c2, table_wide,
                       num_cores=info.num_cores,
                       num_subcores=info.num_subcores)


# speedup vs baseline: 1.5086x; 1.5086x over previous
import functools

import jax
import jax.numpy as jnp
from jax import lax
from jax.experimental import pallas as pl
from jax.experimental.pallas import tpu as pltpu
from jax.experimental.pallas import tpu_sc as plsc

_LANES = 16
_CHUNK = 128
_WIDE = 128


def _widen_rows(table_t):
    d, v = table_t.shape
    cols_per_blk = 2048
    n_blk = (v + cols_per_blk - 1) // cols_per_blk

    def body(in_ref, out_ref):
        out_ref[:, 0:d] = in_ref[...].T

    return pl.pallas_call(
        body,
        grid=(n_blk,),
        in_specs=[pl.BlockSpec((d, cols_per_blk), lambda i: (0, i))],
        out_specs=pl.BlockSpec((cols_per_blk, _WIDE), lambda i: (i, 0)),
        out_shape=jax.ShapeDtypeStruct((v, _WIDE), jnp.float32),
    )(table_t)


@functools.partial(jax.jit, static_argnames=("num_cores", "num_subcores"))
def _w2v_scores(target2d, context2d, table, *, num_cores, num_subcores):
    n_chunks, chunk = target2d.shape
    batch = n_chunks * chunk
    _, wide = table.shape
    embed = 64
    num_workers = num_cores * num_subcores
    b_per_w = batch // num_workers
    chunks_per_w = b_per_w // chunk
    half = b_per_w // 2

    mesh = plsc.VectorSubcoreMesh(core_axis_name="c", subcore_axis_name="s")

    @functools.partial(
        pl.kernel,
        mesh=mesh,
        out_type=jax.ShapeDtypeStruct((batch,), jnp.float32),
        scratch_types=[
            pltpu.VMEM((chunks_per_w, chunk), jnp.int32),
            pltpu.VMEM((chunks_per_w, chunk), jnp.int32),
            pltpu.VMEM((half, wide), jnp.float32),
            pltpu.VMEM((half, wide), jnp.float32),
            pltpu.VMEM((b_per_w,), jnp.float32),
            pltpu.SemaphoreType.DMA,
        ],
        compiler_params=pltpu.CompilerParams(
            needs_layout_passes=False, use_tc_tiling_on_sc=True),
    )
    def k(tgt_hbm, ctx_hbm, table_hbm, out_hbm, tidx_v, cidx_v, trows_v,
          crows_v, out_v, sem):
        wid = lax.axis_index("s") * num_cores + lax.axis_index("c")
        base = wid * b_per_w
        cbase = wid * chunks_per_w

        pltpu.sync_copy(tgt_hbm.at[pl.ds(cbase, chunks_per_w)], tidx_v)
        pltpu.sync_copy(ctx_hbm.at[pl.ds(cbase, chunks_per_w)], cidx_v)

        lane_iota = lax.iota(jnp.int32, _LANES)
        n_sub = embed // _LANES
        chunks_per_half = half // chunk

        for h in range(2):
            copies = []
            for j in range(chunks_per_half):
                cj = h * chunks_per_half + j
                copies.append(pltpu.async_copy(
                    table_hbm.at[tidx_v.at[cj]],
                    trows_v.at[pl.ds(j * chunk, chunk)], sem))
                copies.append(pltpu.async_copy(
                    table_hbm.at[cidx_v.at[cj]],
                    crows_v.at[pl.ds(j * chunk, chunk)], sem))
            for c in copies:
                c.wait()

            def group_body(g, _):
                base_r = g * _LANES
                res = jnp.zeros((_LANES,), jnp.float32)
                for r in range(_LANES):
                    row = base_r + r
                    acc = (trows_v[row, pl.ds(0, _LANES)] *
                           crows_v[row, pl.ds(0, _LANES)])
                    for j in range(1, n_sub):
                        acc = acc + (trows_v[row, pl.ds(j * _LANES, _LANES)] *
                                     crows_v[row, pl.ds(j * _LANES, _LANES)])
                    res = jnp.where(lane_iota == r, jnp.sum(acc), res)
                out_v[pl.ds(h * half + base_r, _LANES)] = res
                return 0

            lax.fori_loop(0, half // _LANES, group_body, 0)

        pltpu.sync_copy(out_v, out_hbm.at[pl.ds(base, b_per_w)])

    return k(target2d, context2d, table)


def kernel(target, context, word_embeddings):
    info = plsc.get_sparse_core_info()
    batch = target.shape[0]
    t2 = target.reshape(batch // _CHUNK, _CHUNK)
    c2 = context.reshape(batch // _CHUNK, _CHUNK)
    table_wide = _widen_rows(word_embeddings.T)
    return _w2v_scores(t2, c2, table_wide,
                       num_cores=info.num_cores,
                       num_subcores=info.num_subcores)


# speedup vs baseline: 2.0051x; 1.3291x over previous
import functools

import jax
import jax.numpy as jnp
from jax import lax
from jax.experimental import pallas as pl
from jax.experimental.pallas import tpu as pltpu
from jax.experimental.pallas import tpu_sc as plsc

_LANES = 16
_CHUNK = 128
_WIDE = 128


def _widen_rows(table_t):
    d, v = table_t.shape
    cols_per_blk = 8192
    n_blk = (v + cols_per_blk - 1) // cols_per_blk

    def body(in_ref, out_ref):
        out_ref[:, 0:d] = in_ref[...].T

    return pl.pallas_call(
        body,
        grid=(n_blk,),
        in_specs=[pl.BlockSpec((d, cols_per_blk), lambda i: (0, i))],
        out_specs=pl.BlockSpec((cols_per_blk, _WIDE), lambda i: (i, 0)),
        out_shape=jax.ShapeDtypeStruct((v, _WIDE), jnp.float32),
    )(table_t)


@functools.partial(jax.jit, static_argnames=("num_cores", "num_subcores"))
def _w2v_scores(target2d, context2d, table, *, num_cores, num_subcores):
    n_chunks, chunk = target2d.shape
    batch = n_chunks * chunk
    _, wide = table.shape
    embed = 64
    num_workers = num_cores * num_subcores
    b_per_w = batch // num_workers
    chunks_per_w = b_per_w // chunk
    half = b_per_w // 2

    mesh = plsc.VectorSubcoreMesh(core_axis_name="c", subcore_axis_name="s")

    @functools.partial(
        pl.kernel,
        mesh=mesh,
        out_type=jax.ShapeDtypeStruct((batch,), jnp.float32),
        scratch_types=[
            pltpu.VMEM((chunks_per_w, chunk), jnp.int32),
            pltpu.VMEM((chunks_per_w, chunk), jnp.int32),
            pltpu.VMEM((half, wide), jnp.float32),
            pltpu.VMEM((half, wide), jnp.float32),
            pltpu.VMEM((b_per_w,), jnp.float32),
            pltpu.SemaphoreType.DMA,
        ],
        compiler_params=pltpu.CompilerParams(
            needs_layout_passes=False, use_tc_tiling_on_sc=True),
    )
    def k(tgt_hbm, ctx_hbm, table_hbm, out_hbm, tidx_v, cidx_v, trows_v,
          crows_v, out_v, sem):
        wid = lax.axis_index("s") * num_cores + lax.axis_index("c")
        base = wid * b_per_w
        cbase = wid * chunks_per_w

        pltpu.sync_copy(tgt_hbm.at[pl.ds(cbase, chunks_per_w)], tidx_v)
        pltpu.sync_copy(ctx_hbm.at[pl.ds(cbase, chunks_per_w)], cidx_v)

        lane_iota = lax.iota(jnp.int32, _LANES)
        n_sub = embed // _LANES
        chunks_per_half = half // chunk

        for h in range(2):
            copies = []
            for j in range(chunks_per_half):
                cj = h * chunks_per_half + j
                copies.append(pltpu.async_copy(
                    table_hbm.at[tidx_v.at[cj]],
                    trows_v.at[pl.ds(j * chunk, chunk)], sem))
                copies.append(pltpu.async_copy(
                    table_hbm.at[cidx_v.at[cj]],
                    crows_v.at[pl.ds(j * chunk, chunk)], sem))
            for c in copies:
                c.wait()

            def group_body(g, _):
                base_r = g * _LANES
                res = jnp.zeros((_LANES,), jnp.float32)
                for r in range(_LANES):
                    row = base_r + r
                    acc = (trows_v[row, pl.ds(0, _LANES)] *
                           crows_v[row, pl.ds(0, _LANES)])
                    for j in range(1, n_sub):
                        acc = acc + (trows_v[row, pl.ds(j * _LANES, _LANES)] *
                                     crows_v[row, pl.ds(j * _LANES, _LANES)])
                    res = jnp.where(lane_iota == r, jnp.sum(acc), res)
                out_v[pl.ds(h * half + base_r, _LANES)] = res
                return 0

            lax.fori_loop(0, half // _LANES, group_body, 0)

        pltpu.sync_copy(out_v, out_hbm.at[pl.ds(base, b_per_w)])

    return k(target2d, context2d, table)


def kernel(target, context, word_embeddings):
    info = plsc.get_sparse_core_info()
    batch = target.shape[0]
    t2 = target.reshape(batch // _CHUNK, _CHUNK)
    c2 = context.reshape(batch // _CHUNK, _CHUNK)
    table_wide = _widen_rows(word_embeddings.T)
    return _w2v_scores(t2, c2, table_wide,
                       num_cores=info.num_cores,
                       num_subcores=info.num_subcores)


# speedup vs baseline: 2.0793x; 1.0370x over previous
import functools

import jax
import jax.numpy as jnp
from jax import lax
from jax.experimental import pallas as pl
from jax.experimental.pallas import tpu as pltpu
from jax.experimental.pallas import tpu_sc as plsc

_LANES = 16
_CHUNK = 128
_WIDE = 128


def _widen_rows(table_t):
    d, v = table_t.shape
    cols_per_blk = 25600
    n_blk = (v + cols_per_blk - 1) // cols_per_blk

    def body(in_ref, out_ref):
        out_ref[:, 0:d] = in_ref[...].T

    return pl.pallas_call(
        body,
        grid=(n_blk,),
        in_specs=[pl.BlockSpec((d, cols_per_blk), lambda i: (0, i))],
        out_specs=pl.BlockSpec((cols_per_blk, _WIDE), lambda i: (i, 0)),
        out_shape=jax.ShapeDtypeStruct((v, _WIDE), jnp.float32),
    )(table_t)


@functools.partial(jax.jit, static_argnames=("num_cores", "num_subcores"))
def _w2v_scores(target2d, context2d, table, *, num_cores, num_subcores):
    n_chunks, chunk = target2d.shape
    batch = n_chunks * chunk
    _, wide = table.shape
    embed = 64
    num_workers = num_cores * num_subcores
    b_per_w = batch // num_workers
    chunks_per_w = b_per_w // chunk
    half = b_per_w // 2

    mesh = plsc.VectorSubcoreMesh(core_axis_name="c", subcore_axis_name="s")

    @functools.partial(
        pl.kernel,
        mesh=mesh,
        out_type=jax.ShapeDtypeStruct((batch,), jnp.float32),
        scratch_types=[
            pltpu.VMEM((chunks_per_w, chunk), jnp.int32),
            pltpu.VMEM((chunks_per_w, chunk), jnp.int32),
            pltpu.VMEM((half, wide), jnp.float32),
            pltpu.VMEM((half, wide), jnp.float32),
            pltpu.VMEM((b_per_w,), jnp.float32),
            pltpu.SemaphoreType.DMA,
        ],
        compiler_params=pltpu.CompilerParams(
            needs_layout_passes=False, use_tc_tiling_on_sc=True),
    )
    def k(tgt_hbm, ctx_hbm, table_hbm, out_hbm, tidx_v, cidx_v, trows_v,
          crows_v, out_v, sem):
        wid = lax.axis_index("s") * num_cores + lax.axis_index("c")
        base = wid * b_per_w
        cbase = wid * chunks_per_w

        pltpu.sync_copy(tgt_hbm.at[pl.ds(cbase, chunks_per_w)], tidx_v)
        pltpu.sync_copy(ctx_hbm.at[pl.ds(cbase, chunks_per_w)], cidx_v)

        lane_iota = lax.iota(jnp.int32, _LANES)
        n_sub = embed // _LANES
        chunks_per_half = half // chunk

        for h in range(2):
            copies = []
            for j in range(chunks_per_half):
                cj = h * chunks_per_half + j
                copies.append(pltpu.async_copy(
                    table_hbm.at[tidx_v.at[cj]],
                    trows_v.at[pl.ds(j * chunk, chunk)], sem))
                copies.append(pltpu.async_copy(
                    table_hbm.at[cidx_v.at[cj]],
                    crows_v.at[pl.ds(j * chunk, chunk)], sem))
            for c in copies:
                c.wait()

            def group_body(g, _):
                base_r = g * _LANES
                res = jnp.zeros((_LANES,), jnp.float32)
                for r in range(_LANES):
                    row = base_r + r
                    acc = (trows_v[row, pl.ds(0, _LANES)] *
                           crows_v[row, pl.ds(0, _LANES)])
                    for j in range(1, n_sub):
                        acc = acc + (trows_v[row, pl.ds(j * _LANES, _LANES)] *
                                     crows_v[row, pl.ds(j * _LANES, _LANES)])
                    res = jnp.where(lane_iota == r, jnp.sum(acc), res)
                out_v[pl.ds(h * half + base_r, _LANES)] = res
                return 0

            lax.fori_loop(0, half // _LANES, group_body, 0)

        pltpu.sync_copy(out_v, out_hbm.at[pl.ds(base, b_per_w)])

    return k(target2d, context2d, table)


def kernel(target, context, word_embeddings):
    info = plsc.get_sparse_core_info()
    batch = target.shape[0]
    t2 = target.reshape(batch // _CHUNK, _CHUNK)
    c2 = context.reshape(batch // _CHUNK, _CHUNK)
    table_wide = _widen_rows(word_embeddings.T)
    return _w2v_scores(t2, c2, table_wide,
                       num_cores=info.num_cores,
                       num_subcores=info.num_subcores)
